# baseline (speedup 1.0000x reference)
import functools

import jax
import jax.numpy as jnp
import numpy as np
from jax import lax
from jax.experimental import pallas as pl
from jax.experimental.pallas import tpu as pltpu
from jax.experimental.pallas import tpu_sc as plsc

NUM_SYMP = 100000
NUM_DISE = 1000
D = 64
B = 1024
N1 = 20
N2 = 20
HIST = 200

NC, NS = 2, 16
NW = NC * NS
SPW = B // NW
UC0, UC1 = 104, 96

BB = 256
GB = B // BB
RB = BB * N1

_UPERM = np.array([g * 32 + 2 * i + h
                   for g in range(2) for h in range(2) for i in range(16)],
                  dtype=np.int32)


def _sc_body(symp, symp_bf, dise, usu_i, d1_i, d2_i, lbl,
             usu_o, embs_o, sds_o, tgt_o,
             tab, ia_u, ia_1, ia_2, il, ru, r1, rt, su, ss, g0, g1, gd):
  cid = lax.axis_index("c")
  sid = lax.axis_index("s")
  wid = sid * NC + cid
  base = wid * SPW
  sems = (g0, g1)
  f32 = jnp.float32
  i32 = jnp.int32

  pltpu.sync_copy(dise, tab)
  pltpu.sync_copy(usu_i.at[pl.ds(base, SPW)], ia_u)
  pltpu.sync_copy(d1_i.at[pl.ds(base * N1, SPW * N1)], ia_1)
  pltpu.sync_copy(d2_i.at[pl.ds(base, SPW)], ia_2)
  pltpu.sync_copy(lbl.at[pl.ds(base, SPW)], il)
  pltpu.async_copy(dise.at[il], rt, gd).wait()
  pltpu.sync_copy(rt, tgt_o.at[pl.ds(base, SPW)])

  zero = jnp.zeros((16,), f32)
  iota16 = lax.broadcasted_iota(i32, (16,), 0)
  cols = [iota16 + 16 * c for c in range(4)]

  def fire_u(t, s):
    pltpu.async_copy(symp_bf.at[ia_u.at[t, pl.ds(0, UC0)]],
                     ru.at[s].at[pl.ds(0, UC0)], sems[s])
    pltpu.async_copy(symp_bf.at[ia_u.at[t, pl.ds(UC0, UC1)]],
                     ru.at[s].at[pl.ds(UC0, UC1)], sems[s])

  def drain_u(s):
    pltpu.make_async_copy(symp_bf.at[pl.ds(0, UC0)],
                          ru.at[s].at[pl.ds(0, UC0)], sems[s]).wait()
    pltpu.make_async_copy(symp_bf.at[pl.ds(0, UC1)],
                          ru.at[s].at[pl.ds(UC0, UC1)], sems[s]).wait()

  fire_u(0, 0)
  fire_u(1, 1)

  def step(it, carry):
    pltpu.async_copy(symp.at[ia_1.at[pl.ds(it * 80, 80)]], r1, gd)
    for m in range(4):
      s = m % 2
      t = 4 * it + m
      b = base + t
      drain_u(s)

      def ub(r, a, s=s):
        for dr in range(4):
          row = r * 4 + dr
          for g in range(2):
            x = ru[s, row, pl.ds(g * 32, 32)]
            ev, od = plsc.unpack(x, format=plsc.PackFormat.INTERLEAVED)
            if g == 0:
              a = (a[0] + ev, a[1] + od, a[2], a[3])
            else:
              a = (a[0], a[1], a[2] + ev, a[3] + od)
        return a
      acc = lax.fori_loop(0, HIST // 4, ub, (zero, zero, zero, zero))
      for k in range(4):
        su[t, pl.ds(k * 16, 16)] = acc[k]

      fire_u(jnp.minimum(t + 2, SPW - 1), s)

      def db(sg, carry2, t=t):
        accs = (zero, zero, zero, zero)
        for blk in range(5):
          iv = ia_2[t, pl.ds(sg * 80 + blk * 16, 16)]
          for j in range(16):
            r = blk * 16 + j
            seg, pos = divmod(r, N2)
            splat = jnp.broadcast_to(iv[j], (16,))
            vals = [plsc.load_gather(tab, [splat, cols[c]])
                    for c in range(4)]
            accs = tuple(accs[c] + vals[c] for c in range(4))
            if pos == N2 - 1:
              for c in range(4):
                ss[sg * 4 + seg, pl.ds(c * 16, 16)] = accs[c]
              accs = (zero, zero, zero, zero)
        return carry2
      lax.fori_loop(0, 5, db, 0)
      pltpu.sync_copy(ss, sds_o.at[pl.ds(b * N1, N1)])

    pltpu.make_async_copy(symp.at[pl.ds(0, 80)], r1, gd).wait()
    pltpu.sync_copy(r1, embs_o.at[pl.ds(base * N1 + it * 80, 80)])
    return carry

  lax.fori_loop(0, SPW // 4, step, 0)
  drain_u(0)
  drain_u(1)
  pltpu.sync_copy(su, usu_o.at[pl.ds(base, SPW)])


@functools.cache
def _get_sc_gather():
  return pl.kernel(
    _sc_body,
    out_type=[
        jax.ShapeDtypeStruct((B, D), jnp.float32),
        jax.ShapeDtypeStruct((B * N1, D), jnp.float32),
        jax.ShapeDtypeStruct((B * N1, D), jnp.float32),
        jax.ShapeDtypeStruct((B, D), jnp.float32),
    ],
    mesh=plsc.VectorSubcoreMesh(core_axis_name="c", subcore_axis_name="s",
                                num_cores=NC, num_subcores=NS),
    scratch_types=[
        pltpu.VMEM((NUM_DISE + 1, D), jnp.float32),
        pltpu.VMEM((SPW, HIST), jnp.int32),
        pltpu.VMEM((SPW * N1,), jnp.int32),
        pltpu.VMEM((SPW, N1 * N2), jnp.int32),
        pltpu.VMEM((SPW,), jnp.int32),
        pltpu.VMEM((2, HIST, D), jnp.bfloat16),
        pltpu.VMEM((80, D), jnp.float32),
        pltpu.VMEM((SPW, D), jnp.float32),
        pltpu.VMEM((SPW, D), jnp.float32),
        pltpu.VMEM((N1, D), jnp.float32),
        pltpu.SemaphoreType.DMA,
        pltpu.SemaphoreType.DMA,
        pltpu.SemaphoreType.DMA,
    ],
    compiler_params=pltpu.CompilerParams(use_tc_tiling_on_sc=False,
                                         needs_layout_passes=False),
  )


def _inv_cnt(cnt):
  w = 1.0 / (cnt + 1e-8)
  return jnp.where(w >= 1e8, 0.0, w)


def _tc_body(d1, d2, uu, usum, embs, sds, tgt,
             w_u, w21, w22, w11, w12, out):
  c2 = jnp.sum((d2[...] != 0).astype(jnp.float32), axis=1, keepdims=True)
  sd_avg = sds[...] * _inv_cnt(c2)
  es = embs[...]
  f32 = jnp.float32
  t = (jnp.dot(es + sd_avg, w21[...], preferred_element_type=f32)
       + jnp.dot(sd_avg * es, w22[...], preferred_element_type=f32))
  h = jnp.tanh(t)
  nrm = jnp.sqrt(jnp.sum(h * h, axis=1, keepdims=True))
  e1 = h / jnp.maximum(nrm, 1e-12)

  tg = tgt[...]
  tg_rep = jnp.broadcast_to(tg[:, None, :], (BB, N1, D)).reshape(RB, D)
  msg = (jnp.dot(e1, w11[...], preferred_element_type=f32)
         + jnp.dot(e1 * tg_rep, w12[...], preferred_element_type=f32))
  pooled = jnp.sum(msg.reshape(BB, N1, D), axis=1)
  c1 = jnp.sum((d1[...] != 0).astype(jnp.float32), axis=1, keepdims=True)
  emb_dise = jnp.tanh(pooled * _inv_cnt(c1)
                      + jnp.dot(tg, w11[...], preferred_element_type=f32))

  cu = jnp.sum((uu[...] != 0).astype(jnp.float32), axis=1, keepdims=True)
  emb_user = jnp.tanh(jnp.dot(usum[...] * _inv_cnt(cu), w_u[...],
                              preferred_element_type=f32))
  out[...] = jnp.sum(emb_dise * emb_user, axis=1)


def _tc_dense(d1, d2r, uu, usum, embs, sds, tgt, w_u, w21, w22, w11, w12,
              interpret=False):
  wspec = pl.BlockSpec((D, D), lambda i: (0, 0))
  return pl.pallas_call(
      _tc_body,
      grid=(GB,),
      in_specs=[
          pl.BlockSpec((BB, N1), lambda i: (i, 0)),
          pl.BlockSpec((RB, N2), lambda i: (i, 0)),
          pl.BlockSpec((BB, HIST), lambda i: (i, 0)),
          pl.BlockSpec((BB, D), lambda i: (i, 0)),
          pl.BlockSpec((RB, D), lambda i: (i, 0)),
          pl.BlockSpec((RB, D), lambda i: (i, 0)),
          pl.BlockSpec((BB, D), lambda i: (i, 0)),
          wspec, wspec, wspec, wspec, wspec,
      ],
      out_specs=pl.BlockSpec((BB,), lambda i: (i,)),
      out_shape=jax.ShapeDtypeStruct((B,), jnp.float32),
      interpret=interpret,
  )(d1, d2r, uu, usum, embs, sds, tgt, w_u, w21, w22, w11, w12)


@jax.jit
def kernel(label, dsd_1, dsd_2, usu_1, symp_table, dise_table,
           W_usu1, W_dsd_2_1, W_dsd_2_2, W_dsd_1_1, W_dsd_1_2):
  if label.dtype != jnp.int32:
    label = label.astype(jnp.int32)
  if dsd_1.dtype != jnp.int32:
    dsd_1 = dsd_1.astype(jnp.int32)
  if dsd_2.dtype != jnp.int32:
    dsd_2 = dsd_2.astype(jnp.int32)
  if usu_1.dtype != jnp.int32:
    usu_1 = usu_1.astype(jnp.int32)

  symp_bf = symp_table.astype(jnp.bfloat16)
  w_u_perm = W_usu1[jnp.asarray(_UPERM), :]

  usum, embs, sds, tgt = _get_sc_gather()(
      symp_table, symp_bf, dise_table,
      usu_1, dsd_1.reshape(B * N1), dsd_2.reshape(B, N1 * N2), label)
  score = _tc_dense(dsd_1, dsd_2.reshape(B * N1, N2), usu_1, usum,
                    embs, sds, tgt,
                    w_u_perm, W_dsd_2_1, W_dsd_2_2, W_dsd_1_1, W_dsd_1_2)
  return score


# speedup vs baseline: 1.0165x; 1.0165x over previous
import functools

import jax
import jax.numpy as jnp
import numpy as np
from jax import lax
from jax.experimental import pallas as pl
from jax.experimental.pallas import tpu as pltpu
from jax.experimental.pallas import tpu_sc as plsc

NUM_SYMP = 100000
NUM_DISE = 1000
D = 64
B = 1024
N1 = 20
N2 = 20
HIST = 200

NC, NS = 2, 16
NW = NC * NS
SPW = B // NW
UC0, UC1 = 104, 96

BB = 256
GB = B // BB
RB = BB * N1

_UPERM = np.array([g * 32 + 2 * i + h
                   for g in range(2) for h in range(2) for i in range(16)],
                  dtype=np.int32)


def _sc_body(symp, symp_bf, dise, usu_i, d1_i, d2_i, lbl,
             usu_o, embs_o, sds_o, tgt_o,
             tab, ia_u, ia_1, ia_2, il, ru, r1, rt, su, ss, g0, g1, gd):
  cid = lax.axis_index("c")
  sid = lax.axis_index("s")
  wid = sid * NC + cid
  base = wid * SPW
  sems = (g0, g1)
  f32 = jnp.float32
  i32 = jnp.int32

  pltpu.sync_copy(dise, tab)
  pltpu.sync_copy(usu_i.at[pl.ds(base, SPW)], ia_u)
  pltpu.sync_copy(d1_i.at[pl.ds(base * N1, SPW * N1)], ia_1)
  pltpu.sync_copy(d2_i.at[pl.ds(base, SPW)], ia_2)
  pltpu.sync_copy(lbl.at[pl.ds(base, SPW)], il)
  pltpu.async_copy(dise.at[il], rt, gd).wait()
  pltpu.sync_copy(rt, tgt_o.at[pl.ds(base, SPW)])

  zero = jnp.zeros((16,), f32)
  iota16 = lax.broadcasted_iota(i32, (16,), 0)
  cols = [iota16 + 16 * c for c in range(4)]

  def fire_u(t, s):
    pltpu.async_copy(symp_bf.at[ia_u.at[t, pl.ds(0, UC0)]],
                     ru.at[s].at[pl.ds(0, UC0)], sems[s])
    pltpu.async_copy(symp_bf.at[ia_u.at[t, pl.ds(UC0, UC1)]],
                     ru.at[s].at[pl.ds(UC0, UC1)], sems[s])

  def drain_u(s):
    pltpu.make_async_copy(symp_bf.at[pl.ds(0, UC0)],
                          ru.at[s].at[pl.ds(0, UC0)], sems[s]).wait()
    pltpu.make_async_copy(symp_bf.at[pl.ds(0, UC1)],
                          ru.at[s].at[pl.ds(UC0, UC1)], sems[s]).wait()

  fire_u(0, 0)
  fire_u(1, 1)

  def step(it, carry):
    pltpu.async_copy(symp.at[ia_1.at[pl.ds(it * 80, 80)]], r1, gd)
    for m in range(4):
      s = m % 2
      t = 4 * it + m
      b = base + t
      drain_u(s)

      def ub(r, a, s=s):
        for dr in range(4):
          row = r * 4 + dr
          for g in range(2):
            x = ru[s, row, pl.ds(g * 32, 32)]
            ev, od = plsc.unpack(x, format=plsc.PackFormat.INTERLEAVED)
            if g == 0:
              a = (a[0] + ev, a[1] + od, a[2], a[3])
            else:
              a = (a[0], a[1], a[2] + ev, a[3] + od)
        return a
      acc = lax.fori_loop(0, HIST // 4, ub, (zero, zero, zero, zero))
      for k in range(4):
        su[t, pl.ds(k * 16, 16)] = acc[k]

      fire_u(jnp.minimum(t + 2, SPW - 1), s)

      def db(sg, carry2, t=t):
        accs = (zero, zero, zero, zero)
        for blk in range(5):
          iv = ia_2[t, pl.ds(sg * 80 + blk * 16, 16)]
          for j in range(16):
            r = blk * 16 + j
            seg, pos = divmod(r, N2)
            splat = jnp.broadcast_to(iv[j], (16,))
            vals = [plsc.load_gather(tab, [splat, cols[c]])
                    for c in range(4)]
            accs = tuple(accs[c] + vals[c] for c in range(4))
            if pos == N2 - 1:
              for c in range(4):
                ss[sg * 4 + seg, pl.ds(c * 16, 16)] = accs[c]
              accs = (zero, zero, zero, zero)
        return carry2
      lax.fori_loop(0, 5, db, 0)
      pltpu.sync_copy(ss, sds_o.at[pl.ds(b * N1, N1)])

    pltpu.make_async_copy(symp.at[pl.ds(0, 80)], r1, gd).wait()
    pltpu.sync_copy(r1, embs_o.at[pl.ds(base * N1 + it * 80, 80)])
    return carry

  lax.fori_loop(0, SPW // 4, step, 0)
  drain_u(0)
  drain_u(1)
  pltpu.sync_copy(su, usu_o.at[pl.ds(base, SPW)])


@functools.cache
def _get_sc_gather():
  return pl.kernel(
    _sc_body,
    out_type=[
        jax.ShapeDtypeStruct((B, D), jnp.float32),
        jax.ShapeDtypeStruct((B * N1, D), jnp.float32),
        jax.ShapeDtypeStruct((B * N1, D), jnp.float32),
        jax.ShapeDtypeStruct((B, D), jnp.float32),
    ],
    mesh=plsc.VectorSubcoreMesh(core_axis_name="c", subcore_axis_name="s",
                                num_cores=NC, num_subcores=NS),
    scratch_types=[
        pltpu.VMEM((NUM_DISE + 1, D), jnp.float32),
        pltpu.VMEM((SPW, HIST), jnp.int32),
        pltpu.VMEM((SPW * N1,), jnp.int32),
        pltpu.VMEM((SPW, N1 * N2), jnp.int32),
        pltpu.VMEM((SPW,), jnp.int32),
        pltpu.VMEM((2, HIST, D), jnp.bfloat16),
        pltpu.VMEM((80, D), jnp.float32),
        pltpu.VMEM((SPW, D), jnp.float32),
        pltpu.VMEM((SPW, D), jnp.float32),
        pltpu.VMEM((N1, D), jnp.float32),
        pltpu.SemaphoreType.DMA,
        pltpu.SemaphoreType.DMA,
        pltpu.SemaphoreType.DMA,
    ],
    compiler_params=pltpu.CompilerParams(use_tc_tiling_on_sc=False,
                                         needs_layout_passes=False),
  )


_CAST_ROWS = 12512


def _cast_body(src, dst):
  dst[...] = src[...].astype(jnp.bfloat16)


def _tc_cast(symp):
  return pl.pallas_call(
      _cast_body,
      grid=(8,),
      in_specs=[pl.BlockSpec((_CAST_ROWS, D), lambda i: (i, 0))],
      out_specs=pl.BlockSpec((_CAST_ROWS, D), lambda i: (i, 0)),
      out_shape=jax.ShapeDtypeStruct((8 * _CAST_ROWS, D), jnp.bfloat16),
  )(symp)


def _inv_cnt(cnt):
  w = 1.0 / (cnt + 1e-8)
  return jnp.where(w >= 1e8, 0.0, w)


def _tc_body(d1, d2, uu, usum, embs, sds, tgt,
             w_u, w21, w22, w11, w12, out):
  c2 = jnp.sum((d2[...] != 0).astype(jnp.float32), axis=1, keepdims=True)
  sd_avg = sds[...] * _inv_cnt(c2)
  es = embs[...]
  f32 = jnp.float32
  t = (jnp.dot(es + sd_avg, w21[...], preferred_element_type=f32)
       + jnp.dot(sd_avg * es, w22[...], preferred_element_type=f32))
  h = jnp.tanh(t)
  nrm = jnp.sqrt(jnp.sum(h * h, axis=1, keepdims=True))
  e1 = h / jnp.maximum(nrm, 1e-12)

  tg = tgt[...]
  tg_rep = jnp.broadcast_to(tg[:, None, :], (BB, N1, D)).reshape(RB, D)
  msg = (jnp.dot(e1, w11[...], preferred_element_type=f32)
         + jnp.dot(e1 * tg_rep, w12[...], preferred_element_type=f32))
  pooled = jnp.sum(msg.reshape(BB, N1, D), axis=1)
  c1 = jnp.sum((d1[...] != 0).astype(jnp.float32), axis=1, keepdims=True)
  emb_dise = jnp.tanh(pooled * _inv_cnt(c1)
                      + jnp.dot(tg, w11[...], preferred_element_type=f32))

  cu = jnp.sum((uu[...] != 0).astype(jnp.float32), axis=1, keepdims=True)
  emb_user = jnp.tanh(jnp.dot(usum[...] * _inv_cnt(cu), w_u[...],
                              preferred_element_type=f32))
  out[...] = jnp.sum(emb_dise * emb_user, axis=1)


def _tc_dense(d1, d2r, uu, usum, embs, sds, tgt, w_u, w21, w22, w11, w12,
              interpret=False):
  wspec = pl.BlockSpec((D, D), lambda i: (0, 0))
  return pl.pallas_call(
      _tc_body,
      grid=(GB,),
      in_specs=[
          pl.BlockSpec((BB, N1), lambda i: (i, 0)),
          pl.BlockSpec((RB, N2), lambda i: (i, 0)),
          pl.BlockSpec((BB, HIST), lambda i: (i, 0)),
          pl.BlockSpec((BB, D), lambda i: (i, 0)),
          pl.BlockSpec((RB, D), lambda i: (i, 0)),
          pl.BlockSpec((RB, D), lambda i: (i, 0)),
          pl.BlockSpec((BB, D), lambda i: (i, 0)),
          wspec, wspec, wspec, wspec, wspec,
      ],
      out_specs=pl.BlockSpec((BB,), lambda i: (i,)),
      out_shape=jax.ShapeDtypeStruct((B,), jnp.float32),
      interpret=interpret,
  )(d1, d2r, uu, usum, embs, sds, tgt, w_u, w21, w22, w11, w12)


@jax.jit
def kernel(label, dsd_1, dsd_2, usu_1, symp_table, dise_table,
           W_usu1, W_dsd_2_1, W_dsd_2_2, W_dsd_1_1, W_dsd_1_2):
  if label.dtype != jnp.int32:
    label = label.astype(jnp.int32)
  if dsd_1.dtype != jnp.int32:
    dsd_1 = dsd_1.astype(jnp.int32)
  if dsd_2.dtype != jnp.int32:
    dsd_2 = dsd_2.astype(jnp.int32)
  if usu_1.dtype != jnp.int32:
    usu_1 = usu_1.astype(jnp.int32)

  symp_bf = _tc_cast(symp_table)
  w_u_perm = W_usu1[jnp.asarray(_UPERM), :]

  usum, embs, sds, tgt = _get_sc_gather()(
      symp_table, symp_bf, dise_table,
      usu_1, dsd_1.reshape(B * N1), dsd_2.reshape(B, N1 * N2), label)
  score = _tc_dense(dsd_1, dsd_2.reshape(B * N1, N2), usu_1, usum,
                    embs, sds, tgt,
                    w_u_perm, W_dsd_2_1, W_dsd_2_2, W_dsd_1_1, W_dsd_1_2)
  return score


# speedup vs baseline: 1.1615x; 1.1427x over previous
import functools

import jax
import jax.numpy as jnp
import numpy as np
from jax import lax
from jax.experimental import pallas as pl
from jax.experimental.pallas import tpu as pltpu
from jax.experimental.pallas import tpu_sc as plsc

NUM_SYMP = 100000
NUM_DISE = 1000
D = 64
B = 1024
N1 = 20
N2 = 20
HIST = 200

NC, NS = 2, 16
NW = NC * NS
SPW = B // NW
UC0, UC1 = 104, 96

BB = 256
GB = B // BB
RB = BB * N1

_UPERM = np.array([g * 32 + 2 * i + h
                   for g in range(2) for h in range(2) for i in range(16)],
                  dtype=np.int32)


def _sc_body(symp_bf, dise, usu_i, d1_i, d2_i, lbl,
             usu_o, embs_o, sds_o, tgt_o,
             tab, ia_u, ia_1, ia_2, il, ru, r1, rt, su, ss, g0, g1, gd):
  cid = lax.axis_index("c")
  sid = lax.axis_index("s")
  wid = sid * NC + cid
  base = wid * SPW
  sems = (g0, g1)
  f32 = jnp.float32
  i32 = jnp.int32

  pltpu.sync_copy(dise, tab)
  pltpu.sync_copy(usu_i.at[pl.ds(base, SPW)], ia_u)
  pltpu.sync_copy(d1_i.at[pl.ds(base * N1, SPW * N1)], ia_1)
  pltpu.sync_copy(d2_i.at[pl.ds(base, SPW)], ia_2)
  pltpu.sync_copy(lbl.at[pl.ds(base, SPW)], il)
  pltpu.async_copy(dise.at[il], rt, gd).wait()
  pltpu.sync_copy(rt, tgt_o.at[pl.ds(base, SPW)])

  zero = jnp.zeros((16,), f32)
  iota16 = lax.broadcasted_iota(i32, (16,), 0)
  cols = [iota16 + 16 * c for c in range(4)]

  def fire_u(t, s):
    pltpu.async_copy(symp_bf.at[ia_u.at[t, pl.ds(0, UC0)]],
                     ru.at[s].at[pl.ds(0, UC0)], sems[s])
    pltpu.async_copy(symp_bf.at[ia_u.at[t, pl.ds(UC0, UC1)]],
                     ru.at[s].at[pl.ds(UC0, UC1)], sems[s])

  def drain_u(s):
    pltpu.make_async_copy(symp_bf.at[pl.ds(0, UC0)],
                          ru.at[s].at[pl.ds(0, UC0)], sems[s]).wait()
    pltpu.make_async_copy(symp_bf.at[pl.ds(0, UC1)],
                          ru.at[s].at[pl.ds(UC0, UC1)], sems[s]).wait()

  fire_u(0, 0)
  fire_u(1, 1)

  def step(it, carry):
    pltpu.async_copy(symp_bf.at[ia_1.at[pl.ds(it * 80, 80)]], r1, gd)
    for m in range(4):
      s = m % 2
      t = 4 * it + m
      b = base + t
      drain_u(s)

      def ub(r, a, s=s):
        for dr in range(4):
          row = r * 4 + dr
          for g in range(2):
            x = ru[s, row, pl.ds(g * 32, 32)]
            ev, od = plsc.unpack(x, format=plsc.PackFormat.INTERLEAVED)
            if g == 0:
              a = (a[0] + ev, a[1] + od, a[2], a[3])
            else:
              a = (a[0], a[1], a[2] + ev, a[3] + od)
        return a
      acc = lax.fori_loop(0, HIST // 4, ub, (zero, zero, zero, zero))
      for k in range(4):
        su[t, pl.ds(k * 16, 16)] = acc[k]

      fire_u(jnp.minimum(t + 2, SPW - 1), s)

      def db(sg, carry2, t=t):
        accs = (zero, zero, zero, zero)
        for blk in range(5):
          iv = ia_2[t, pl.ds(sg * 80 + blk * 16, 16)]
          for j in range(16):
            r = blk * 16 + j
            seg, pos = divmod(r, N2)
            splat = jnp.broadcast_to(iv[j], (16,))
            vals = [plsc.load_gather(tab, [splat, cols[c]])
                    for c in range(4)]
            accs = tuple(accs[c] + vals[c] for c in range(4))
            if pos == N2 - 1:
              for c in range(4):
                ss[sg * 4 + seg, pl.ds(c * 16, 16)] = accs[c]
              accs = (zero, zero, zero, zero)
        return carry2
      lax.fori_loop(0, 5, db, 0)
      pltpu.sync_copy(ss, sds_o.at[pl.ds(b * N1, N1)])

    pltpu.make_async_copy(symp_bf.at[pl.ds(0, 80)], r1, gd).wait()
    pltpu.sync_copy(r1, embs_o.at[pl.ds(base * N1 + it * 80, 80)])
    return carry

  lax.fori_loop(0, SPW // 4, step, 0)
  drain_u(0)
  drain_u(1)
  pltpu.sync_copy(su, usu_o.at[pl.ds(base, SPW)])


@functools.cache
def _get_sc_gather():
  return pl.kernel(
    _sc_body,
    out_type=[
        jax.ShapeDtypeStruct((B, D), jnp.float32),
        jax.ShapeDtypeStruct((B * N1, D), jnp.bfloat16),
        jax.ShapeDtypeStruct((B * N1, D), jnp.float32),
        jax.ShapeDtypeStruct((B, D), jnp.float32),
    ],
    mesh=plsc.VectorSubcoreMesh(core_axis_name="c", subcore_axis_name="s",
                                num_cores=NC, num_subcores=NS),
    scratch_types=[
        pltpu.VMEM((NUM_DISE + 1, D), jnp.float32),
        pltpu.VMEM((SPW, HIST), jnp.int32),
        pltpu.VMEM((SPW * N1,), jnp.int32),
        pltpu.VMEM((SPW, N1 * N2), jnp.int32),
        pltpu.VMEM((SPW,), jnp.int32),
        pltpu.VMEM((2, HIST, D), jnp.bfloat16),
        pltpu.VMEM((80, D), jnp.bfloat16),
        pltpu.VMEM((SPW, D), jnp.float32),
        pltpu.VMEM((SPW, D), jnp.float32),
        pltpu.VMEM((N1, D), jnp.float32),
        pltpu.SemaphoreType.DMA,
        pltpu.SemaphoreType.DMA,
        pltpu.SemaphoreType.DMA,
    ],
    compiler_params=pltpu.CompilerParams(use_tc_tiling_on_sc=False,
                                         needs_layout_passes=False),
  )


_CAST_ROWS = 12512


def _cast_body(src, dst):
  dst[...] = src[...].astype(jnp.bfloat16)


def _tc_cast(symp):
  return pl.pallas_call(
      _cast_body,
      grid=(8,),
      in_specs=[pl.BlockSpec((_CAST_ROWS, D), lambda i: (i, 0))],
      out_specs=pl.BlockSpec((_CAST_ROWS, D), lambda i: (i, 0)),
      out_shape=jax.ShapeDtypeStruct((8 * _CAST_ROWS, D), jnp.bfloat16),
  )(symp)


def _inv_cnt(cnt):
  w = 1.0 / (cnt + 1e-8)
  return jnp.where(w >= 1e8, 0.0, w)


def _tc_body(d1, d2, uu, usum, embs, sds, tgt,
             w_u, w21, w22, w11, w12, out):
  c2 = jnp.sum((d2[...] != 0).astype(jnp.float32), axis=1, keepdims=True)
  sd_avg = sds[...] * _inv_cnt(c2)
  es = embs[...].astype(jnp.float32)
  f32 = jnp.float32
  t = (jnp.dot(es + sd_avg, w21[...], preferred_element_type=f32)
       + jnp.dot(sd_avg * es, w22[...], preferred_element_type=f32))
  h = jnp.tanh(t)
  nrm = jnp.sqrt(jnp.sum(h * h, axis=1, keepdims=True))
  e1 = h / jnp.maximum(nrm, 1e-12)

  tg = tgt[...]
  tg_rep = jnp.broadcast_to(tg[:, None, :], (BB, N1, D)).reshape(RB, D)
  msg = (jnp.dot(e1, w11[...], preferred_element_type=f32)
         + jnp.dot(e1 * tg_rep, w12[...], preferred_element_type=f32))
  pooled = jnp.sum(msg.reshape(BB, N1, D), axis=1)
  c1 = jnp.sum((d1[...] != 0).astype(jnp.float32), axis=1, keepdims=True)
  emb_dise = jnp.tanh(pooled * _inv_cnt(c1)
                      + jnp.dot(tg, w11[...], preferred_element_type=f32))

  cu = jnp.sum((uu[...] != 0).astype(jnp.float32), axis=1, keepdims=True)
  emb_user = jnp.tanh(jnp.dot(usum[...] * _inv_cnt(cu), w_u[...],
                              preferred_element_type=f32))
  out[...] = jnp.sum(emb_dise * emb_user, axis=1)


def _tc_dense(d1, d2r, uu, usum, embs, sds, tgt, w_u, w21, w22, w11, w12,
              interpret=False):
  wspec = pl.BlockSpec((D, D), lambda i: (0, 0))
  return pl.pallas_call(
      _tc_body,
      grid=(GB,),
      in_specs=[
          pl.BlockSpec((BB, N1), lambda i: (i, 0)),
          pl.BlockSpec((RB, N2), lambda i: (i, 0)),
          pl.BlockSpec((BB, HIST), lambda i: (i, 0)),
          pl.BlockSpec((BB, D), lambda i: (i, 0)),
          pl.BlockSpec((RB, D), lambda i: (i, 0)),
          pl.BlockSpec((RB, D), lambda i: (i, 0)),
          pl.BlockSpec((BB, D), lambda i: (i, 0)),
          wspec, wspec, wspec, wspec, wspec,
      ],
      out_specs=pl.BlockSpec((BB,), lambda i: (i,)),
      out_shape=jax.ShapeDtypeStruct((B,), jnp.float32),
      interpret=interpret,
  )(d1, d2r, uu, usum, embs, sds, tgt, w_u, w21, w22, w11, w12)


@jax.jit
def kernel(label, dsd_1, dsd_2, usu_1, symp_table, dise_table,
           W_usu1, W_dsd_2_1, W_dsd_2_2, W_dsd_1_1, W_dsd_1_2):
  if label.dtype != jnp.int32:
    label = label.astype(jnp.int32)
  if dsd_1.dtype != jnp.int32:
    dsd_1 = dsd_1.astype(jnp.int32)
  if dsd_2.dtype != jnp.int32:
    dsd_2 = dsd_2.astype(jnp.int32)
  if usu_1.dtype != jnp.int32:
    usu_1 = usu_1.astype(jnp.int32)

  symp_bf = _tc_cast(symp_table)
  w_u_perm = W_usu1[jnp.asarray(_UPERM), :]

  usum, embs, sds, tgt = _get_sc_gather()(
      symp_bf, dise_table,
      usu_1, dsd_1.reshape(B * N1), dsd_2.reshape(B, N1 * N2), label)
  score = _tc_dense(dsd_1, dsd_2.reshape(B * N1, N2), usu_1, usum,
                    embs, sds, tgt,
                    w_u_perm, W_dsd_2_1, W_dsd_2_2, W_dsd_1_1, W_dsd_1_2)
  return score


# speedup vs baseline: 1.1671x; 1.0048x over previous
import functools

import jax
import jax.numpy as jnp
import numpy as np
from jax import lax
from jax.experimental import pallas as pl
from jax.experimental.pallas import tpu as pltpu
from jax.experimental.pallas import tpu_sc as plsc

NUM_SYMP = 100000
NUM_DISE = 1000
D = 64
B = 1024
N1 = 20
N2 = 20
HIST = 200

NC, NS = 2, 16
NW = NC * NS
SPW = B // NW
UC0, UC1 = 104, 96

BB = 256
GB = B // BB
RB = BB * N1

_UPERM = np.array([g * 32 + 2 * i + h
                   for g in range(2) for h in range(2) for i in range(16)],
                  dtype=np.int32)


def _sc_body(symp_bf, dise, usu_i, d1_i, d2_i, lbl,
             usu_o, embs_o, sds_o, tgt_o,
             tab, ia_u, ia_1, ia_2, il, ru, r1, rt, su, ss, g0, g1, gd):
  cid = lax.axis_index("c")
  sid = lax.axis_index("s")
  wid = sid * NC + cid
  base = wid * SPW
  sems = (g0, g1)
  f32 = jnp.float32
  i32 = jnp.int32

  pltpu.sync_copy(dise, tab)
  pltpu.sync_copy(usu_i.at[pl.ds(base, SPW)], ia_u)
  pltpu.sync_copy(d1_i.at[pl.ds(base * N1, SPW * N1)], ia_1)
  pltpu.sync_copy(d2_i.at[pl.ds(base, SPW)], ia_2)
  pltpu.sync_copy(lbl.at[pl.ds(base, SPW)], il)
  pltpu.async_copy(dise.at[il], rt, gd).wait()
  pltpu.sync_copy(rt, tgt_o.at[pl.ds(base, SPW)])

  zero = jnp.zeros((16,), f32)
  iota16 = lax.broadcasted_iota(i32, (16,), 0)
  cols = [iota16 + 16 * c for c in range(4)]

  def fire_u(t, s):
    pltpu.async_copy(symp_bf.at[ia_u.at[t, pl.ds(0, UC0)]],
                     ru.at[s].at[pl.ds(0, UC0)], sems[s])
    pltpu.async_copy(symp_bf.at[ia_u.at[t, pl.ds(UC0, UC1)]],
                     ru.at[s].at[pl.ds(UC0, UC1)], sems[s])

  def drain_u(s):
    pltpu.make_async_copy(symp_bf.at[pl.ds(0, UC0)],
                          ru.at[s].at[pl.ds(0, UC0)], sems[s]).wait()
    pltpu.make_async_copy(symp_bf.at[pl.ds(0, UC1)],
                          ru.at[s].at[pl.ds(UC0, UC1)], sems[s]).wait()

  fire_u(0, 0)
  fire_u(1, 1)

  def step(it, carry):
    pltpu.async_copy(symp_bf.at[ia_1.at[pl.ds(it * 80, 80)]], r1, gd)
    for m in range(4):
      s = m % 2
      t = 4 * it + m
      b = base + t
      drain_u(s)

      def ub(r, a, s=s):
        for dr in range(4):
          row = r * 4 + dr
          for g in range(2):
            x = ru[s, row, pl.ds(g * 32, 32)]
            ev, od = plsc.unpack(x, format=plsc.PackFormat.INTERLEAVED)
            if g == 0:
              a = (a[0] + ev, a[1] + od, a[2], a[3])
            else:
              a = (a[0], a[1], a[2] + ev, a[3] + od)
        return a
      acc = lax.fori_loop(0, HIST // 4, ub, (zero, zero, zero, zero))
      for k in range(4):
        su[t, pl.ds(k * 16, 16)] = acc[k]

      fire_u(jnp.minimum(t + 2, SPW - 1), s)

      def db(sg, carry2, t=t, m=m):
        accs = (zero, zero, zero, zero)
        for blk in range(5):
          iv = ia_2[t, pl.ds(sg * 80 + blk * 16, 16)]
          for j in range(16):
            r = blk * 16 + j
            seg, pos = divmod(r, N2)
            splat = jnp.broadcast_to(iv[j], (16,))
            vals = [plsc.load_gather(tab, [splat, cols[c]])
                    for c in range(4)]
            accs = tuple(accs[c] + vals[c] for c in range(4))
            if pos == N2 - 1:
              for c in range(4):
                ss[m * N1 + sg * 4 + seg, pl.ds(c * 16, 16)] = accs[c]
              accs = (zero, zero, zero, zero)
        return carry2
      lax.fori_loop(0, 5, db, 0)

    pltpu.sync_copy(ss, sds_o.at[pl.ds(base * N1 + it * 80, 80)])
    pltpu.make_async_copy(symp_bf.at[pl.ds(0, 80)], r1, gd).wait()
    pltpu.sync_copy(r1, embs_o.at[pl.ds(base * N1 + it * 80, 80)])
    return carry

  lax.fori_loop(0, SPW // 4, step, 0)
  drain_u(0)
  drain_u(1)
  pltpu.sync_copy(su, usu_o.at[pl.ds(base, SPW)])


@functools.cache
def _get_sc_gather():
  return pl.kernel(
    _sc_body,
    out_type=[
        jax.ShapeDtypeStruct((B, D), jnp.float32),
        jax.ShapeDtypeStruct((B * N1, D), jnp.bfloat16),
        jax.ShapeDtypeStruct((B * N1, D), jnp.float32),
        jax.ShapeDtypeStruct((B, D), jnp.float32),
    ],
    mesh=plsc.VectorSubcoreMesh(core_axis_name="c", subcore_axis_name="s",
                                num_cores=NC, num_subcores=NS),
    scratch_types=[
        pltpu.VMEM((NUM_DISE + 1, D), jnp.float32),
        pltpu.VMEM((SPW, HIST), jnp.int32),
        pltpu.VMEM((SPW * N1,), jnp.int32),
        pltpu.VMEM((SPW, N1 * N2), jnp.int32),
        pltpu.VMEM((SPW,), jnp.int32),
        pltpu.VMEM((2, HIST, D), jnp.bfloat16),
        pltpu.VMEM((80, D), jnp.bfloat16),
        pltpu.VMEM((SPW, D), jnp.float32),
        pltpu.VMEM((SPW, D), jnp.float32),
        pltpu.VMEM((4 * N1, D), jnp.float32),
        pltpu.SemaphoreType.DMA,
        pltpu.SemaphoreType.DMA,
        pltpu.SemaphoreType.DMA,
    ],
    compiler_params=pltpu.CompilerParams(use_tc_tiling_on_sc=False,
                                         needs_layout_passes=False),
  )


_CAST_ROWS = 12512


def _cast_body(src, dst):
  dst[...] = src[...].astype(jnp.bfloat16)


def _tc_cast(symp):
  return pl.pallas_call(
      _cast_body,
      grid=(8,),
      in_specs=[pl.BlockSpec((_CAST_ROWS, D), lambda i: (i, 0))],
      out_specs=pl.BlockSpec((_CAST_ROWS, D), lambda i: (i, 0)),
      out_shape=jax.ShapeDtypeStruct((8 * _CAST_ROWS, D), jnp.bfloat16),
  )(symp)


def _inv_cnt(cnt):
  w = 1.0 / (cnt + 1e-8)
  return jnp.where(w >= 1e8, 0.0, w)


def _tc_body(d1, d2, uu, usum, embs, sds, tgt,
             w_u, w21, w22, w11, w12, out):
  c2 = jnp.sum((d2[...] != 0).astype(jnp.float32), axis=1, keepdims=True)
  sd_avg = sds[...] * _inv_cnt(c2)
  es = embs[...].astype(jnp.float32)
  f32 = jnp.float32
  t = (jnp.dot(es + sd_avg, w21[...], preferred_element_type=f32)
       + jnp.dot(sd_avg * es, w22[...], preferred_element_type=f32))
  h = jnp.tanh(t)
  nrm = jnp.sqrt(jnp.sum(h * h, axis=1, keepdims=True))
  e1 = h / jnp.maximum(nrm, 1e-12)

  tg = tgt[...]
  tg_rep = jnp.broadcast_to(tg[:, None, :], (BB, N1, D)).reshape(RB, D)
  msg = (jnp.dot(e1, w11[...], preferred_element_type=f32)
         + jnp.dot(e1 * tg_rep, w12[...], preferred_element_type=f32))
  pooled = jnp.sum(msg.reshape(BB, N1, D), axis=1)
  c1 = jnp.sum((d1[...] != 0).astype(jnp.float32), axis=1, keepdims=True)
  emb_dise = jnp.tanh(pooled * _inv_cnt(c1)
                      + jnp.dot(tg, w11[...], preferred_element_type=f32))

  cu = jnp.sum((uu[...] != 0).astype(jnp.float32), axis=1, keepdims=True)
  emb_user = jnp.tanh(jnp.dot(usum[...] * _inv_cnt(cu), w_u[...],
                              preferred_element_type=f32))
  out[...] = jnp.sum(emb_dise * emb_user, axis=1)


def _tc_dense(d1, d2r, uu, usum, embs, sds, tgt, w_u, w21, w22, w11, w12,
              interpret=False):
  wspec = pl.BlockSpec((D, D), lambda i: (0, 0))
  return pl.pallas_call(
      _tc_body,
      grid=(GB,),
      in_specs=[
          pl.BlockSpec((BB, N1), lambda i: (i, 0)),
          pl.BlockSpec((RB, N2), lambda i: (i, 0)),
          pl.BlockSpec((BB, HIST), lambda i: (i, 0)),
          pl.BlockSpec((BB, D), lambda i: (i, 0)),
          pl.BlockSpec((RB, D), lambda i: (i, 0)),
          pl.BlockSpec((RB, D), lambda i: (i, 0)),
          pl.BlockSpec((BB, D), lambda i: (i, 0)),
          wspec, wspec, wspec, wspec, wspec,
      ],
      out_specs=pl.BlockSpec((BB,), lambda i: (i,)),
      out_shape=jax.ShapeDtypeStruct((B,), jnp.float32),
      interpret=interpret,
  )(d1, d2r, uu, usum, embs, sds, tgt, w_u, w21, w22, w11, w12)


@jax.jit
def kernel(label, dsd_1, dsd_2, usu_1, symp_table, dise_table,
           W_usu1, W_dsd_2_1, W_dsd_2_2, W_dsd_1_1, W_dsd_1_2):
  if label.dtype != jnp.int32:
    label = label.astype(jnp.int32)
  if dsd_1.dtype != jnp.int32:
    dsd_1 = dsd_1.astype(jnp.int32)
  if dsd_2.dtype != jnp.int32:
    dsd_2 = dsd_2.astype(jnp.int32)
  if usu_1.dtype != jnp.int32:
    usu_1 = usu_1.astype(jnp.int32)

  symp_bf = _tc_cast(symp_table)
  w_u_perm = W_usu1[jnp.asarray(_UPERM), :]

  usum, embs, sds, tgt = _get_sc_gather()(
      symp_bf, dise_table,
      usu_1, dsd_1.reshape(B * N1), dsd_2.reshape(B, N1 * N2), label)
  score = _tc_dense(dsd_1, dsd_2.reshape(B * N1, N2), usu_1, usum,
                    embs, sds, tgt,
                    w_u_perm, W_dsd_2_1, W_dsd_2_2, W_dsd_1_1, W_dsd_1_2)
  return score
